# Initial kernel scaffold; baseline (speedup 1.0000x reference)
import jax
import jax.numpy as jnp
from jax.experimental import pallas as pl


def kernel(words, lengths, emb, W_ih, W_hh, b_ih, b_hh):
    raise NotImplementedError("write your pallas kernel here")


# speedup vs baseline: 9.5255x; 9.5255x over previous
import jax
import jax.numpy as jnp
from jax.experimental import pallas as pl
from jax.experimental.pallas import tpu as pltpu

B = 64
T = 512
D = 256
TC = 8


def _lstm_kernel(ids_ref,
                 emb_hbm,
                 wx_ref,
                 wh_ref,
                 bias_ref,
                 lenb_ref,
                 hall_ref,
                 fin_ref,
                 emb_v,
                 copy_sem,
                 xa, xb,
                 h_s, c_s):
    d = pl.program_id(0)
    tc = pl.program_id(1)

    @pl.when(tc == 0)
    def _init():
        cp = pltpu.make_async_copy(emb_hbm, emb_v, copy_sem)
        cp.start()
        cp.wait()
        h_s[...] = jnp.zeros_like(h_s)
        c_s[...] = jnp.zeros_like(c_s)

    wx = wx_ref[...]
    wh = wh_ref[...]
    bias = bias_ref[...]
    lenb = lenb_ref[...]
    base = d * (T * B) + tc * (TC * B)

    for s in range(TC):
        t_idx = tc * TC + s
        for mi in range(B):
            idx = ids_ref[base + s * B + mi]
            i2 = pl.multiple_of(idx * 2, 2)
            slab = emb_v[pl.ds(i2, 2), :]
            xa[pl.ds(mi, 1), :] = slab[0:1, :]
            xb[pl.ds(mi, 1), :] = slab[1:2, :]
        x = jnp.concatenate([xa[...], xb[...]], axis=-1)

        g = (jnp.dot(x, wx, preferred_element_type=jnp.float32)
             + jnp.dot(h_s[...], wh, preferred_element_type=jnp.float32)
             + bias)
        gi = g[:, 0:D]
        gf = g[:, D:2 * D]
        gg = g[:, 2 * D:3 * D]
        go = g[:, 3 * D:4 * D]
        c2 = jax.nn.sigmoid(gf) * c_s[...] + jax.nn.sigmoid(gi) * jnp.tanh(gg)
        h2 = jax.nn.sigmoid(go) * jnp.tanh(c2)
        m = lenb > t_idx
        hn = jnp.where(m, h2, h_s[...])
        h_s[...] = hn
        c_s[...] = jnp.where(m, c2, c_s[...])
        hall_ref[s] = jnp.where(m, hn, 0.0)

    fin_ref[0] = h_s[...]


def kernel(words, lengths, emb, W_ih, W_hh, b_ih, b_hh):
    V = emb.shape[0]
    lengths = lengths.astype(jnp.int32)
    words = words.astype(jnp.int32)

    idx_rev = jnp.clip(lengths[:, None] - 1 - jnp.arange(T)[None, :], 0)
    words_rev = jnp.take_along_axis(words, idx_rev, axis=1)
    ids_all = jnp.stack([words.T, words_rev.T])
    ids_flat = ids_all.reshape(-1)

    emb2 = emb.reshape(V * 2, 128)
    wx = W_ih.T
    wh = W_hh.T
    biasv = (b_ih + b_hh).reshape(1, 4 * D)
    lenb = jnp.broadcast_to(lengths[:, None], (B, D))

    grid = (2, T // TC)
    hall, finals = pl.pallas_call(
        _lstm_kernel,
        grid_spec=pltpu.PrefetchScalarGridSpec(
            num_scalar_prefetch=1,
            grid=grid,
            in_specs=[
                pl.BlockSpec(memory_space=pl.ANY),
                pl.BlockSpec((D, 4 * D), lambda d, tc, ids: (0, 0)),
                pl.BlockSpec((D, 4 * D), lambda d, tc, ids: (0, 0)),
                pl.BlockSpec((1, 4 * D), lambda d, tc, ids: (0, 0)),
                pl.BlockSpec((B, D), lambda d, tc, ids: (0, 0)),
            ],
            out_specs=[
                pl.BlockSpec((TC, B, D), lambda d, tc, ids: (tc, 0, d)),
                pl.BlockSpec((1, B, D), lambda d, tc, ids: (d, 0, 0)),
            ],
            scratch_shapes=[
                pltpu.VMEM((V * 2, 128), jnp.float32),
                pltpu.SemaphoreType.DMA,
                pltpu.VMEM((B, 128), jnp.float32),
                pltpu.VMEM((B, 128), jnp.float32),
                pltpu.VMEM((B, D), jnp.float32),
                pltpu.VMEM((B, D), jnp.float32),
            ],
        ),
        out_shape=[
            jax.ShapeDtypeStruct((T, B, 2 * D), jnp.float32),
            jax.ShapeDtypeStruct((2, B, D), jnp.float32),
        ],
        compiler_params=pltpu.CompilerParams(
            dimension_semantics=("parallel", "arbitrary"),
            vmem_limit_bytes=56 * 1024 * 1024,
        ),
        name="bidir_lstm_encoder",
    )(ids_flat, emb2, wx, wh, biasv, lenb)

    out = finals[1, B - 1:B, :]
    return (out, hall)


# speedup vs baseline: 11.8820x; 1.2474x over previous
import jax
import jax.numpy as jnp
from jax.experimental import pallas as pl
from jax.experimental.pallas import tpu as pltpu

B = 64
T = 512
D = 256
TC = 8
M = 2 * B * TC
S = M + 1


def _gather_dot(ids_ref, emb_v, xs, gx_ref, wx, bias, chunk, sel):
    base = chunk * M
    for mi in range(M):
        i2 = pl.multiple_of(ids_ref[base + mi], 2)
        xs[mi:mi + 2 * S:S, :] = emb_v[pl.ds(i2, 2), :]
    x = jnp.concatenate([xs[pl.ds(0, M), :], xs[pl.ds(S, M), :]], axis=-1)
    gx_ref[sel] = (jnp.dot(x, wx, preferred_element_type=jnp.float32)
                   + bias)


def _lstm_kernel(ids_ref,
                 emb_hbm,
                 wx_ref,
                 wh_ref,
                 bias_ref,
                 lenb_ref,
                 hall_ref,
                 fin_ref,
                 emb_v,
                 copy_sem,
                 xs,
                 gx_ref,
                 h_s, c_s):
    k = pl.program_id(0)
    nt = T // TC
    wx = wx_ref[...]
    wh = wh_ref[...]
    bias = bias_ref[...]
    lenb = lenb_ref[...]

    @pl.when(k == 0)
    def _init():
        cp = pltpu.make_async_copy(emb_hbm, emb_v, copy_sem)
        cp.start()
        cp.wait()
        h_s[...] = jnp.zeros_like(h_s)
        c_s[...] = jnp.zeros_like(c_s)

    @pl.when(k < nt)
    def _prefetch():
        _gather_dot(ids_ref, emb_v, xs, gx_ref, wx, bias, k, k & 1)

    @pl.when(k > 0)
    def _chain():
        sel = (k - 1) & 1
        for s in range(TC):
            t_idx = (k - 1) * TC + s
            g = (jnp.dot(h_s[...], wh, preferred_element_type=jnp.float32)
                 + gx_ref[sel, pl.ds(s * 2 * B, 2 * B), :])
            gi = g[:, 0:D]
            gf = g[:, D:2 * D]
            gg = g[:, 2 * D:3 * D]
            go = g[:, 3 * D:4 * D]
            c2 = (jax.nn.sigmoid(gf) * c_s[...]
                  + jax.nn.sigmoid(gi) * jnp.tanh(gg))
            h2 = jax.nn.sigmoid(go) * jnp.tanh(c2)
            m = lenb > t_idx
            hn = jnp.where(m, h2, h_s[...])
            h_s[...] = hn
            c_s[...] = jnp.where(m, c2, c_s[...])
            mo = jnp.where(m, hn, 0.0)
            hall_ref[s, :, 0:D] = mo[0:B]
            hall_ref[s, :, D:2 * D] = mo[B:2 * B]
        fin_ref[...] = h_s[B:2 * B]


def kernel(words, lengths, emb, W_ih, W_hh, b_ih, b_hh):
    V = emb.shape[0]
    lengths = lengths.astype(jnp.int32)
    words = words.astype(jnp.int32)

    idx_rev = jnp.clip(lengths[:, None] - 1 - jnp.arange(T)[None, :], 0)
    words_rev = jnp.take_along_axis(words, idx_rev, axis=1)
    ids_all = jnp.concatenate([words.T, words_rev.T], axis=1)
    ids_flat = ids_all.reshape(-1) * 2

    emb2 = emb.reshape(V * 2, 128)
    wx = W_ih.T
    wh = W_hh.T
    biasv = (b_ih + b_hh).reshape(1, 4 * D)
    lenb = jnp.broadcast_to(lengths[:, None], (B, D))
    lenb2 = jnp.concatenate([lenb, lenb], axis=0)

    nt = T // TC
    hall, finals = pl.pallas_call(
        _lstm_kernel,
        grid_spec=pltpu.PrefetchScalarGridSpec(
            num_scalar_prefetch=1,
            grid=(nt + 1,),
            in_specs=[
                pl.BlockSpec(memory_space=pl.ANY),
                pl.BlockSpec((D, 4 * D), lambda k, ids: (0, 0)),
                pl.BlockSpec((D, 4 * D), lambda k, ids: (0, 0)),
                pl.BlockSpec((1, 4 * D), lambda k, ids: (0, 0)),
                pl.BlockSpec((2 * B, D), lambda k, ids: (0, 0)),
            ],
            out_specs=[
                pl.BlockSpec((TC, B, 2 * D),
                             lambda k, ids: (jnp.maximum(k - 1, 0), 0, 0)),
                pl.BlockSpec((B, D), lambda k, ids: (0, 0)),
            ],
            scratch_shapes=[
                pltpu.VMEM((V * 2, 128), jnp.float32),
                pltpu.SemaphoreType.DMA,
                pltpu.VMEM((2 * S, 128), jnp.float32),
                pltpu.VMEM((2, M, 4 * D), jnp.float32),
                pltpu.VMEM((2 * B, D), jnp.float32),
                pltpu.VMEM((2 * B, D), jnp.float32),
            ],
        ),
        out_shape=[
            jax.ShapeDtypeStruct((T, B, 2 * D), jnp.float32),
            jax.ShapeDtypeStruct((B, D), jnp.float32),
        ],
        compiler_params=pltpu.CompilerParams(
            dimension_semantics=("arbitrary",),
            vmem_limit_bytes=56 * 1024 * 1024,
        ),
        name="bidir_lstm_encoder",
    )(ids_flat, emb2, wx, wh, biasv, lenb2)

    out = finals[B - 1:B, :]
    return (out, hall)


# speedup vs baseline: 12.4225x; 1.0455x over previous
import jax
import jax.numpy as jnp
from jax.experimental import pallas as pl
from jax.experimental.pallas import tpu as pltpu

B = 64
T = 512
D = 256
TC = 8
M = 2 * B * TC
S = M + 1


def _sig(x):
    return 0.5 * jnp.tanh(0.5 * x) + 0.5


def _gather_dot(ids_ref, emb_v, xs, gx_ref, wx, bias, chunk, sel):
    base = chunk * M
    for mi in range(M):
        i2 = pl.multiple_of(ids_ref[base + mi], 2)
        xs[mi:mi + 2 * S:S, :] = emb_v[pl.ds(i2, 2), :]
    for j in range(M // 128):
        xj = jnp.concatenate([xs[pl.ds(j * 128, 128), :],
                              xs[pl.ds(S + j * 128, 128), :]], axis=-1)
        gx_ref[sel, pl.ds(j * 128, 128), :] = (
            jnp.dot(xj, wx, preferred_element_type=jnp.float32) + bias)


def _lstm_kernel(ids_ref,
                 emb_hbm,
                 wx_ref,
                 wh_ref,
                 bias_ref,
                 lenb_ref,
                 hall_ref,
                 fin_ref,
                 emb_v,
                 copy_sem,
                 xs,
                 gx_ref,
                 h_s, c_s):
    k = pl.program_id(0)
    nt = T // TC
    wx = wx_ref[...]
    wh = wh_ref[...]
    bias = bias_ref[...]
    lenb = lenb_ref[...]

    @pl.when(k == 0)
    def _init():
        cp = pltpu.make_async_copy(emb_hbm, emb_v, copy_sem)
        cp.start()
        cp.wait()
        h_s[...] = jnp.zeros_like(h_s)
        c_s[...] = jnp.zeros_like(c_s)

    @pl.when(k < nt)
    def _prefetch():
        _gather_dot(ids_ref, emb_v, xs, gx_ref, wx, bias, k, k & 1)

    @pl.when(k > 0)
    def _chain():
        sel = (k - 1) & 1
        for s in range(TC):
            t_idx = (k - 1) * TC + s
            g = (jnp.dot(h_s[...], wh, preferred_element_type=jnp.float32)
                 + gx_ref[sel, pl.ds(s * 2 * B, 2 * B), :])
            gi = g[:, 0:D]
            gf = g[:, D:2 * D]
            gg = g[:, 2 * D:3 * D]
            go = g[:, 3 * D:4 * D]
            c2 = _sig(gf) * c_s[...] + _sig(gi) * jnp.tanh(gg)
            h2 = _sig(go) * jnp.tanh(c2)
            m = lenb > t_idx
            hn = jnp.where(m, h2, h_s[...])
            h_s[...] = hn
            c_s[...] = jnp.where(m, c2, c_s[...])
            mo = jnp.where(m, hn, 0.0)
            hall_ref[s, :, 0:D] = mo[0:B]
            hall_ref[s, :, D:2 * D] = mo[B:2 * B]
        fin_ref[...] = h_s[B:2 * B]


def kernel(words, lengths, emb, W_ih, W_hh, b_ih, b_hh):
    V = emb.shape[0]
    lengths = lengths.astype(jnp.int32)
    words = words.astype(jnp.int32)

    idx_rev = jnp.clip(lengths[:, None] - 1 - jnp.arange(T)[None, :], 0)
    words_rev = jnp.take_along_axis(words, idx_rev, axis=1)
    ids_all = jnp.concatenate([words.T, words_rev.T], axis=1)
    ids_flat = ids_all.reshape(-1) * 2

    emb2 = emb.reshape(V * 2, 128)
    wx = W_ih.T
    wh = W_hh.T
    biasv = (b_ih + b_hh).reshape(1, 4 * D)
    lenb = jnp.broadcast_to(lengths[:, None], (B, D))
    lenb2 = jnp.concatenate([lenb, lenb], axis=0)

    nt = T // TC
    hall, finals = pl.pallas_call(
        _lstm_kernel,
        grid_spec=pltpu.PrefetchScalarGridSpec(
            num_scalar_prefetch=1,
            grid=(nt + 1,),
            in_specs=[
                pl.BlockSpec(memory_space=pl.ANY),
                pl.BlockSpec((D, 4 * D), lambda k, ids: (0, 0)),
                pl.BlockSpec((D, 4 * D), lambda k, ids: (0, 0)),
                pl.BlockSpec((1, 4 * D), lambda k, ids: (0, 0)),
                pl.BlockSpec((2 * B, D), lambda k, ids: (0, 0)),
            ],
            out_specs=[
                pl.BlockSpec((TC, B, 2 * D),
                             lambda k, ids: (jnp.maximum(k - 1, 0), 0, 0)),
                pl.BlockSpec((B, D), lambda k, ids: (0, 0)),
            ],
            scratch_shapes=[
                pltpu.VMEM((V * 2, 128), jnp.float32),
                pltpu.SemaphoreType.DMA,
                pltpu.VMEM((2 * S, 128), jnp.float32),
                pltpu.VMEM((2, M, 4 * D), jnp.float32),
                pltpu.VMEM((2 * B, D), jnp.float32),
                pltpu.VMEM((2 * B, D), jnp.float32),
            ],
        ),
        out_shape=[
            jax.ShapeDtypeStruct((T, B, 2 * D), jnp.float32),
            jax.ShapeDtypeStruct((B, D), jnp.float32),
        ],
        compiler_params=pltpu.CompilerParams(
            dimension_semantics=("arbitrary",),
            vmem_limit_bytes=56 * 1024 * 1024,
        ),
        name="bidir_lstm_encoder",
    )(ids_flat, emb2, wx, wh, biasv, lenb2)

    out = finals[B - 1:B, :]
    return (out, hall)


# speedup vs baseline: 12.9381x; 1.0415x over previous
import jax
import jax.numpy as jnp
from jax.experimental import pallas as pl
from jax.experimental.pallas import tpu as pltpu

B = 64
T = 512
D = 256
TC = 8
M = 2 * B * TC
S = M + 1


def _sig(x):
    return 0.5 * jnp.tanh(0.5 * x) + 0.5


def _gather_dot(ids_ref, emb_v, xs, gx_ref, wx, bias, chunk, sel):
    base = chunk * TC
    for s in range(TC):
        for r in range(2 * B):
            mi = s * 2 * B + r
            i2 = pl.multiple_of(ids_ref[r * T + base + s], 2)
            xs[mi:mi + 2 * S:S, :] = emb_v[pl.ds(i2, 2), :]
    for j in range(M // 128):
        xj = jnp.concatenate([xs[pl.ds(j * 128, 128), :],
                              xs[pl.ds(S + j * 128, 128), :]], axis=-1)
        gx_ref[sel, pl.ds(j * 128, 128), :] = (
            jnp.dot(xj, wx, preferred_element_type=jnp.float32) + bias)


def _lstm_kernel(ids_ref,
                 emb_hbm,
                 wih_ref,
                 whh_ref,
                 bias_ref,
                 lenb_ref,
                 hall_ref,
                 fin_ref,
                 emb_v,
                 copy_sem,
                 xs,
                 gx_ref,
                 wx_s, wh_s,
                 h_s, c_s):
    k = pl.program_id(0)
    nt = T // TC
    bias = bias_ref[...]
    lenb = lenb_ref[...]

    @pl.when(k == 0)
    def _init():
        cp = pltpu.make_async_copy(emb_hbm, emb_v, copy_sem)
        cp.start()
        wx_s[...] = wih_ref[...].T
        wh_s[...] = whh_ref[...].T
        h_s[...] = jnp.zeros_like(h_s)
        c_s[...] = jnp.zeros_like(c_s)
        cp.wait()

    @pl.when(k < nt)
    def _prefetch():
        _gather_dot(ids_ref, emb_v, xs, gx_ref, wx_s[...], bias, k, k & 1)

    @pl.when(k > 0)
    def _chain():
        sel = (k - 1) & 1
        wh = wh_s[...]
        for s in range(TC):
            t_idx = (k - 1) * TC + s
            g = (jnp.dot(h_s[...], wh, preferred_element_type=jnp.float32)
                 + gx_ref[sel, pl.ds(s * 2 * B, 2 * B), :])
            gi = g[:, 0:D]
            gf = g[:, D:2 * D]
            gg = g[:, 2 * D:3 * D]
            go = g[:, 3 * D:4 * D]
            c2 = _sig(gf) * c_s[...] + _sig(gi) * jnp.tanh(gg)
            h2 = _sig(go) * jnp.tanh(c2)
            m = lenb > t_idx
            hn = jnp.where(m, h2, h_s[...])
            h_s[...] = hn
            c_s[...] = jnp.where(m, c2, c_s[...])
            mo = jnp.where(m, hn, 0.0)
            hall_ref[s, :, 0:D] = mo[0:B]
            hall_ref[s, :, D:2 * D] = mo[B:2 * B]
        fin_ref[...] = h_s[2 * B - 1:2 * B, :]


def kernel(words, lengths, emb, W_ih, W_hh, b_ih, b_hh):
    V = emb.shape[0]
    lengths = lengths.astype(jnp.int32)
    words = words.astype(jnp.int32)

    idx_rev = jnp.clip(lengths[:, None] - 1 - jnp.arange(T)[None, :], 0)
    words_rev = jnp.take_along_axis(words, idx_rev, axis=1)
    ids_all = jnp.concatenate([words, words_rev], axis=0)
    ids_flat = ids_all.reshape(-1) * 2

    emb2 = emb.reshape(V * 2, 128)
    biasv = (b_ih + b_hh).reshape(1, 4 * D)
    lenb = jnp.broadcast_to(lengths[:, None], (B, D))
    lenb2 = jnp.concatenate([lenb, lenb], axis=0)

    nt = T // TC
    hall, finals = pl.pallas_call(
        _lstm_kernel,
        grid_spec=pltpu.PrefetchScalarGridSpec(
            num_scalar_prefetch=1,
            grid=(nt + 1,),
            in_specs=[
                pl.BlockSpec(memory_space=pl.ANY),
                pl.BlockSpec((4 * D, D), lambda k, ids: (0, 0)),
                pl.BlockSpec((4 * D, D), lambda k, ids: (0, 0)),
                pl.BlockSpec((1, 4 * D), lambda k, ids: (0, 0)),
                pl.BlockSpec((2 * B, D), lambda k, ids: (0, 0)),
            ],
            out_specs=[
                pl.BlockSpec((TC, B, 2 * D),
                             lambda k, ids: (jnp.maximum(k - 1, 0), 0, 0)),
                pl.BlockSpec((1, D), lambda k, ids: (0, 0)),
            ],
            scratch_shapes=[
                pltpu.VMEM((V * 2, 128), jnp.float32),
                pltpu.SemaphoreType.DMA,
                pltpu.VMEM((2 * S, 128), jnp.float32),
                pltpu.VMEM((2, M, 4 * D), jnp.float32),
                pltpu.VMEM((D, 4 * D), jnp.float32),
                pltpu.VMEM((D, 4 * D), jnp.float32),
                pltpu.VMEM((2 * B, D), jnp.float32),
                pltpu.VMEM((2 * B, D), jnp.float32),
            ],
        ),
        out_shape=[
            jax.ShapeDtypeStruct((T, B, 2 * D), jnp.float32),
            jax.ShapeDtypeStruct((1, D), jnp.float32),
        ],
        compiler_params=pltpu.CompilerParams(
            dimension_semantics=("arbitrary",),
            vmem_limit_bytes=56 * 1024 * 1024,
        ),
        name="bidir_lstm_encoder",
    )(ids_flat, emb2, W_ih, W_hh, biasv, lenb2)

    return (finals, hall)


# speedup vs baseline: 13.3271x; 1.0301x over previous
import jax
import jax.numpy as jnp
from jax.experimental import pallas as pl
from jax.experimental.pallas import tpu as pltpu

B = 64
T = 512
D = 256
TC = 8
M = 2 * B * TC
S = M + 1


def _sig(x):
    return 0.5 * jnp.tanh(0.5 * x) + 0.5


def _gather_dot(ids_ref, emb_v, xs, gx_ref, wx, bias, chunk, sel):
    base = chunk * TC
    for s in range(TC):
        for r in range(2 * B):
            mi = s * 2 * B + r
            i2 = pl.multiple_of(ids_ref[r * T + base + s], 2)
            xs[mi:mi + 2 * S:S, :] = emb_v[pl.ds(i2, 2), :]
    for j in range(M // 128):
        xj = jnp.concatenate([xs[pl.ds(j * 128, 128), :],
                              xs[pl.ds(S + j * 128, 128), :]], axis=-1)
        gx_ref[sel, pl.ds(j * 128, 128), :] = (
            jnp.dot(xj, wx, preferred_element_type=jnp.float32) + bias)


def _lstm_kernel(ids_ref,
                 emb_hbm,
                 wih_ref,
                 whh_ref,
                 bias_ref,
                 lenb_ref,
                 hall_ref,
                 fin_ref,
                 emb_v,
                 copy_sem,
                 xs,
                 gx_ref,
                 wx_s, wh_s,
                 h_s, c_s):
    k = pl.program_id(0)
    nt = T // TC
    bias = bias_ref[...]
    lenb = lenb_ref[...]

    @pl.when(k == 0)
    def _init():
        cp = pltpu.make_async_copy(emb_hbm, emb_v, copy_sem)
        cp.start()
        wx_s[...] = wih_ref[...].T
        wh_s[...] = whh_ref[...].T
        h_s[...] = jnp.zeros_like(h_s)
        c_s[...] = jnp.zeros_like(c_s)
        cp.wait()

    chunk = jnp.minimum(k, nt - 1)
    _gather_dot(ids_ref, emb_v, xs, gx_ref, wx_s[...], bias, chunk, k & 1)

    sel = (k - 1) & 1
    tb = jnp.where(k > 0, (k - 1) * TC, 2 * T)
    wh = wh_s[...]
    for s in range(TC):
        t_idx = tb + s
        g = (jnp.dot(h_s[...], wh, preferred_element_type=jnp.float32)
             + gx_ref[sel, pl.ds(s * 2 * B, 2 * B), :])
        gi = g[:, 0:D]
        gf = g[:, D:2 * D]
        gg = g[:, 2 * D:3 * D]
        go = g[:, 3 * D:4 * D]
        c2 = _sig(gf) * c_s[...] + _sig(gi) * jnp.tanh(gg)
        h2 = _sig(go) * jnp.tanh(c2)
        m = lenb > t_idx
        hn = jnp.where(m, h2, h_s[...])
        h_s[...] = hn
        c_s[...] = jnp.where(m, c2, c_s[...])
        mo = jnp.where(m, hn, 0.0)
        hall_ref[s, :, 0:D] = mo[0:B]
        hall_ref[s, :, D:2 * D] = mo[B:2 * B]
    fin_ref[...] = h_s[2 * B - 1:2 * B, :]


def kernel(words, lengths, emb, W_ih, W_hh, b_ih, b_hh):
    V = emb.shape[0]
    lengths = lengths.astype(jnp.int32)
    words = words.astype(jnp.int32)

    idx_rev = jnp.clip(lengths[:, None] - 1 - jnp.arange(T)[None, :], 0)
    words_rev = jnp.take_along_axis(words, idx_rev, axis=1)
    ids_all = jnp.concatenate([words, words_rev], axis=0)
    ids_flat = ids_all.reshape(-1) * 2

    emb2 = emb.reshape(V * 2, 128)
    biasv = (b_ih + b_hh).reshape(1, 4 * D)
    lenb = jnp.broadcast_to(lengths[:, None], (B, D))
    lenb2 = jnp.concatenate([lenb, lenb], axis=0)

    nt = T // TC
    hall, finals = pl.pallas_call(
        _lstm_kernel,
        grid_spec=pltpu.PrefetchScalarGridSpec(
            num_scalar_prefetch=1,
            grid=(nt + 1,),
            in_specs=[
                pl.BlockSpec(memory_space=pl.ANY),
                pl.BlockSpec((4 * D, D), lambda k, ids: (0, 0)),
                pl.BlockSpec((4 * D, D), lambda k, ids: (0, 0)),
                pl.BlockSpec((1, 4 * D), lambda k, ids: (0, 0)),
                pl.BlockSpec((2 * B, D), lambda k, ids: (0, 0)),
            ],
            out_specs=[
                pl.BlockSpec((TC, B, 2 * D),
                             lambda k, ids: (jnp.maximum(k - 1, 0), 0, 0)),
                pl.BlockSpec((1, D), lambda k, ids: (0, 0)),
            ],
            scratch_shapes=[
                pltpu.VMEM((V * 2, 128), jnp.float32),
                pltpu.SemaphoreType.DMA,
                pltpu.VMEM((2 * S, 128), jnp.float32),
                pltpu.VMEM((2, M, 4 * D), jnp.float32),
                pltpu.VMEM((D, 4 * D), jnp.float32),
                pltpu.VMEM((D, 4 * D), jnp.float32),
                pltpu.VMEM((2 * B, D), jnp.float32),
                pltpu.VMEM((2 * B, D), jnp.float32),
            ],
        ),
        out_shape=[
            jax.ShapeDtypeStruct((T, B, 2 * D), jnp.float32),
            jax.ShapeDtypeStruct((1, D), jnp.float32),
        ],
        compiler_params=pltpu.CompilerParams(
            dimension_semantics=("arbitrary",),
            vmem_limit_bytes=56 * 1024 * 1024,
        ),
        name="bidir_lstm_encoder",
    )(ids_flat, emb2, W_ih, W_hh, biasv, lenb2)

    return (finals, hall)


# speedup vs baseline: 13.3754x; 1.0036x over previous
import jax
import jax.numpy as jnp
from jax.experimental import pallas as pl
from jax.experimental.pallas import tpu as pltpu

B = 64
T = 512
D = 256
TC = 8
M = 2 * B * TC
S = M + 1


def _sig(x):
    return 0.5 * jnp.tanh(0.5 * x) + 0.5


def _gather_dot(words_ref, len_ref, emb_v, xs, gx_ref, wx, bias, chunk, sel):
    for s in range(TC):
        tt = chunk * TC + s
        for r in range(2 * B):
            if r < B:
                idx = words_ref[r * T + tt]
            else:
                tr = jnp.maximum(len_ref[r - B] - 1 - tt, 0)
                idx = words_ref[(r - B) * T + tr]
            mi = s * 2 * B + r
            xs[mi:mi + 2 * S:S, :] = emb_v[idx]
    for j in range(M // 128):
        xj = jnp.concatenate([xs[pl.ds(j * 128, 128), :],
                              xs[pl.ds(S + j * 128, 128), :]], axis=-1)
        gx_ref[sel, pl.ds(j * 128, 128), :] = (
            jnp.dot(xj, wx, preferred_element_type=jnp.float32) + bias)


def _lstm_kernel(words_ref,
                 len_ref,
                 emb_hbm,
                 wih_ref,
                 whh_ref,
                 bias_ref,
                 lenb_ref,
                 hall_ref,
                 fin_ref,
                 emb_v,
                 copy_sem,
                 xs,
                 gx_ref,
                 wx_s, wh_s,
                 h_s, c_s):
    k = pl.program_id(0)
    nt = T // TC
    bias = bias_ref[...]
    lenb = lenb_ref[...]

    @pl.when(k == 0)
    def _init():
        c0 = pltpu.make_async_copy(emb_hbm.at[:, 0:128], emb_v.at[:, 0],
                                   copy_sem.at[0])
        c1 = pltpu.make_async_copy(emb_hbm.at[:, 128:256], emb_v.at[:, 1],
                                   copy_sem.at[1])
        c0.start()
        c1.start()
        wx_s[...] = wih_ref[...].T
        wh_s[...] = whh_ref[...].T
        h_s[...] = jnp.zeros_like(h_s)
        c_s[...] = jnp.zeros_like(c_s)
        c0.wait()
        c1.wait()

    chunk = jnp.minimum(k, nt - 1)
    _gather_dot(words_ref, len_ref, emb_v, xs, gx_ref, wx_s[...], bias,
                chunk, k & 1)

    sel = (k - 1) & 1
    tb = jnp.where(k > 0, (k - 1) * TC, 2 * T)
    wh = wh_s[...]
    for s in range(TC):
        t_idx = tb + s
        g = (jnp.dot(h_s[...], wh, preferred_element_type=jnp.float32)
             + gx_ref[sel, pl.ds(s * 2 * B, 2 * B), :])
        gi = g[:, 0:D]
        gf = g[:, D:2 * D]
        gg = g[:, 2 * D:3 * D]
        go = g[:, 3 * D:4 * D]
        c2 = _sig(gf) * c_s[...] + _sig(gi) * jnp.tanh(gg)
        h2 = _sig(go) * jnp.tanh(c2)
        m = lenb > t_idx
        hn = jnp.where(m, h2, h_s[...])
        h_s[...] = hn
        c_s[...] = jnp.where(m, c2, c_s[...])
        mo = jnp.where(m, hn, 0.0)
        hall_ref[s, :, 0:D] = mo[0:B]
        hall_ref[s, :, D:2 * D] = mo[B:2 * B]
    fin_ref[...] = h_s[2 * B - 1:2 * B, :]


def kernel(words, lengths, emb, W_ih, W_hh, b_ih, b_hh):
    lengths = lengths.astype(jnp.int32)
    words_flat = words.astype(jnp.int32).reshape(-1)

    biasv = (b_ih + b_hh).reshape(1, 4 * D)
    lenb = jnp.broadcast_to(lengths[:, None], (B, D))
    lenb2 = jnp.concatenate([lenb, lenb], axis=0)

    V = emb.shape[0]
    nt = T // TC
    hall, finals = pl.pallas_call(
        _lstm_kernel,
        grid_spec=pltpu.PrefetchScalarGridSpec(
            num_scalar_prefetch=2,
            grid=(nt + 1,),
            in_specs=[
                pl.BlockSpec(memory_space=pl.ANY),
                pl.BlockSpec((4 * D, D), lambda k, w, l: (0, 0)),
                pl.BlockSpec((4 * D, D), lambda k, w, l: (0, 0)),
                pl.BlockSpec((1, 4 * D), lambda k, w, l: (0, 0)),
                pl.BlockSpec((2 * B, D), lambda k, w, l: (0, 0)),
            ],
            out_specs=[
                pl.BlockSpec((TC, B, 2 * D),
                             lambda k, w, l: (jnp.maximum(k - 1, 0), 0, 0)),
                pl.BlockSpec((1, D), lambda k, w, l: (0, 0)),
            ],
            scratch_shapes=[
                pltpu.VMEM((V, 2, 128), jnp.float32),
                pltpu.SemaphoreType.DMA((2,)),
                pltpu.VMEM((2 * S, 128), jnp.float32),
                pltpu.VMEM((2, M, 4 * D), jnp.float32),
                pltpu.VMEM((D, 4 * D), jnp.float32),
                pltpu.VMEM((D, 4 * D), jnp.float32),
                pltpu.VMEM((2 * B, D), jnp.float32),
                pltpu.VMEM((2 * B, D), jnp.float32),
            ],
        ),
        out_shape=[
            jax.ShapeDtypeStruct((T, B, 2 * D), jnp.float32),
            jax.ShapeDtypeStruct((1, D), jnp.float32),
        ],
        compiler_params=pltpu.CompilerParams(
            dimension_semantics=("arbitrary",),
            vmem_limit_bytes=56 * 1024 * 1024,
        ),
        name="bidir_lstm_encoder",
    )(words_flat, lengths, emb, W_ih, W_hh, biasv, lenb2)

    return (finals, hall)


# speedup vs baseline: 14.0422x; 1.0499x over previous
import jax
import jax.numpy as jnp
from jax.experimental import pallas as pl
from jax.experimental.pallas import tpu as pltpu

B = 64
T = 512
D = 256
TC = 8
M = 2 * B * TC
S = M + 1


def _sig(x):
    return 0.5 * jnp.tanh(0.5 * x) + 0.5


def _gather_dot(words_ref, wrev_ref, emb_v, xs, gx_ref, wx, bias, chunk, sel):
    for s in range(TC):
        tt = chunk * TC + s
        for r in range(2 * B):
            if r < B:
                idx = words_ref[r * T + tt]
            else:
                idx = wrev_ref[(r - B) * T + tt]
            mi = s * 2 * B + r
            xs[mi:mi + 2 * S:S, :] = emb_v[idx]
    for j in range(M // 128):
        xj = jnp.concatenate([xs[pl.ds(j * 128, 128), :],
                              xs[pl.ds(S + j * 128, 128), :]], axis=-1)
        gx_ref[sel, pl.ds(j * 128, 128), :] = (
            jnp.dot(xj, wx, preferred_element_type=jnp.float32) + bias)


def _lstm_kernel(words_ref,
                 wrev_ref,
                 emb_hbm,
                 wih_ref,
                 whh_ref,
                 bias_ref,
                 lenb_ref,
                 hall_ref,
                 fin_ref,
                 emb_v,
                 copy_sem,
                 xs,
                 gx_ref,
                 wx_s, wh_s,
                 h_s, c_s):
    k = pl.program_id(0)
    nt = T // TC
    bias = bias_ref[...]
    lenb = lenb_ref[...]

    @pl.when(k == 0)
    def _init():
        c0 = pltpu.make_async_copy(emb_hbm.at[:, 0:128], emb_v.at[:, 0],
                                   copy_sem.at[0])
        c1 = pltpu.make_async_copy(emb_hbm.at[:, 128:256], emb_v.at[:, 1],
                                   copy_sem.at[1])
        c0.start()
        c1.start()
        wx_s[...] = wih_ref[...].T
        wh_s[...] = whh_ref[...].T
        h_s[...] = jnp.zeros_like(h_s)
        c_s[...] = jnp.zeros_like(c_s)
        c0.wait()
        c1.wait()

    chunk = jnp.minimum(k, nt - 1)
    _gather_dot(words_ref, wrev_ref, emb_v, xs, gx_ref, wx_s[...], bias,
                chunk, k & 1)

    sel = (k - 1) & 1
    tb = jnp.where(k > 0, (k - 1) * TC, 2 * T)
    wh = wh_s[...]
    for s in range(TC):
        t_idx = tb + s
        g = (jnp.dot(h_s[...], wh, preferred_element_type=jnp.float32)
             + gx_ref[sel, pl.ds(s * 2 * B, 2 * B), :])
        gi = g[:, 0:D]
        gf = g[:, D:2 * D]
        gg = g[:, 2 * D:3 * D]
        go = g[:, 3 * D:4 * D]
        c2 = _sig(gf) * c_s[...] + _sig(gi) * jnp.tanh(gg)
        h2 = _sig(go) * jnp.tanh(c2)
        m = lenb > t_idx
        hn = jnp.where(m, h2, h_s[...])
        h_s[...] = hn
        c_s[...] = jnp.where(m, c2, c_s[...])
        mo = jnp.where(m, hn, 0.0)
        hall_ref[s, :, 0:D] = mo[0:B]
        hall_ref[s, :, D:2 * D] = mo[B:2 * B]
    fin_ref[...] = h_s[2 * B - 1:2 * B, :]


def kernel(words, lengths, emb, W_ih, W_hh, b_ih, b_hh):
    lengths = lengths.astype(jnp.int32)
    words = words.astype(jnp.int32)
    words_flat = words.reshape(-1)
    idx_rev = jnp.clip(lengths[:, None] - 1 - jnp.arange(T)[None, :], 0)
    wrev_flat = jnp.take_along_axis(words, idx_rev, axis=1).reshape(-1)

    biasv = (b_ih + b_hh).reshape(1, 4 * D)
    lenb = jnp.broadcast_to(lengths[:, None], (B, D))
    lenb2 = jnp.concatenate([lenb, lenb], axis=0)

    V = emb.shape[0]
    nt = T // TC
    hall, finals = pl.pallas_call(
        _lstm_kernel,
        grid_spec=pltpu.PrefetchScalarGridSpec(
            num_scalar_prefetch=2,
            grid=(nt + 1,),
            in_specs=[
                pl.BlockSpec(memory_space=pl.ANY),
                pl.BlockSpec((4 * D, D), lambda k, w, l: (0, 0)),
                pl.BlockSpec((4 * D, D), lambda k, w, l: (0, 0)),
                pl.BlockSpec((1, 4 * D), lambda k, w, l: (0, 0)),
                pl.BlockSpec((2 * B, D), lambda k, w, l: (0, 0)),
            ],
            out_specs=[
                pl.BlockSpec((TC, B, 2 * D),
                             lambda k, w, l: (jnp.maximum(k - 1, 0), 0, 0)),
                pl.BlockSpec((1, D), lambda k, w, l: (0, 0)),
            ],
            scratch_shapes=[
                pltpu.VMEM((V, 2, 128), jnp.float32),
                pltpu.SemaphoreType.DMA((2,)),
                pltpu.VMEM((2 * S, 128), jnp.float32),
                pltpu.VMEM((2, M, 4 * D), jnp.float32),
                pltpu.VMEM((D, 4 * D), jnp.float32),
                pltpu.VMEM((D, 4 * D), jnp.float32),
                pltpu.VMEM((2 * B, D), jnp.float32),
                pltpu.VMEM((2 * B, D), jnp.float32),
            ],
        ),
        out_shape=[
            jax.ShapeDtypeStruct((T, B, 2 * D), jnp.float32),
            jax.ShapeDtypeStruct((1, D), jnp.float32),
        ],
        compiler_params=pltpu.CompilerParams(
            dimension_semantics=("arbitrary",),
            vmem_limit_bytes=56 * 1024 * 1024,
        ),
        name="bidir_lstm_encoder",
    )(words_flat, wrev_flat, emb, W_ih, W_hh, biasv, lenb2)

    return (finals, hall)


# speedup vs baseline: 14.2330x; 1.0136x over previous
import jax
import jax.numpy as jnp
from jax.experimental import pallas as pl
from jax.experimental.pallas import tpu as pltpu

B = 64
T = 512
D = 256
TC = 8
M = 2 * B * TC
S = M + 1


def _sig(x):
    return 0.5 * jnp.tanh(0.5 * x) + 0.5


def _gather_dot(words_ref, wrev_ref, emb_v, xs, gx_ref, wx, bias, chunk, sel):
    for s in range(TC):
        tt = chunk * TC + s
        for r in range(2 * B):
            if r < B:
                idx = words_ref[r * T + tt]
            else:
                idx = wrev_ref[(r - B) * T + tt]
            mi = s * 2 * B + r
            xs[mi:mi + 2 * S:S, :] = emb_v[idx]
    for j in range(M // 512):
        xj = jnp.concatenate([xs[pl.ds(j * 512, 512), :],
                              xs[pl.ds(S + j * 512, 512), :]], axis=-1)
        gx_ref[sel, pl.ds(j * 512, 512), :] = (
            jnp.dot(xj, wx, preferred_element_type=jnp.float32) + bias)


def _lstm_kernel(words_ref,
                 wrev_ref,
                 emb_hbm,
                 wih_ref,
                 whh_ref,
                 bias_ref,
                 lenb_ref,
                 hall_ref,
                 fin_ref,
                 emb_v,
                 copy_sem,
                 xs,
                 gx_ref,
                 wx_s, wh_s,
                 h_s, c_s):
    k = pl.program_id(0)
    nt = T // TC
    bias = bias_ref[...]
    lenb = lenb_ref[...]

    @pl.when(k == 0)
    def _init():
        c0 = pltpu.make_async_copy(emb_hbm.at[:, 0:128], emb_v.at[:, 0],
                                   copy_sem.at[0])
        c1 = pltpu.make_async_copy(emb_hbm.at[:, 128:256], emb_v.at[:, 1],
                                   copy_sem.at[1])
        c0.start()
        c1.start()
        wx_s[...] = wih_ref[...].T
        wh_s[...] = whh_ref[...].T
        h_s[...] = jnp.zeros_like(h_s)
        c_s[...] = jnp.zeros_like(c_s)
        c0.wait()
        c1.wait()

    chunk = jnp.minimum(k, nt - 1)
    _gather_dot(words_ref, wrev_ref, emb_v, xs, gx_ref, wx_s[...], bias,
                chunk, k & 1)

    sel = (k - 1) & 1
    tb = jnp.where(k > 0, (k - 1) * TC, 2 * T)
    wh = wh_s[...]
    for s in range(TC):
        t_idx = tb + s
        g = (jnp.dot(h_s[...], wh, preferred_element_type=jnp.float32)
             + gx_ref[sel, pl.ds(s * 2 * B, 2 * B), :])
        gi = g[:, 0:D]
        gf = g[:, D:2 * D]
        gg = g[:, 2 * D:3 * D]
        go = g[:, 3 * D:4 * D]
        c2 = _sig(gf) * c_s[...] + _sig(gi) * jnp.tanh(gg)
        h2 = _sig(go) * jnp.tanh(c2)
        m = lenb > t_idx
        hn = jnp.where(m, h2, h_s[...])
        h_s[...] = hn
        c_s[...] = jnp.where(m, c2, c_s[...])
        mo = jnp.where(m, hn, 0.0)
        hall_ref[s, :, 0:D] = mo[0:B]
        hall_ref[s, :, D:2 * D] = mo[B:2 * B]
    fin_ref[...] = h_s[2 * B - 1:2 * B, :]


def kernel(words, lengths, emb, W_ih, W_hh, b_ih, b_hh):
    lengths = lengths.astype(jnp.int32)
    words = words.astype(jnp.int32)
    words_flat = words.reshape(-1)
    idx_rev = jnp.clip(lengths[:, None] - 1 - jnp.arange(T)[None, :], 0)
    wrev_flat = jnp.take_along_axis(words, idx_rev, axis=1).reshape(-1)

    biasv = (b_ih + b_hh).reshape(1, 4 * D)
    lenb = jnp.broadcast_to(lengths[:, None], (B, D))
    lenb2 = jnp.concatenate([lenb, lenb], axis=0)

    V = emb.shape[0]
    nt = T // TC
    hall, finals = pl.pallas_call(
        _lstm_kernel,
        grid_spec=pltpu.PrefetchScalarGridSpec(
            num_scalar_prefetch=2,
            grid=(nt + 1,),
            in_specs=[
                pl.BlockSpec(memory_space=pl.ANY),
                pl.BlockSpec((4 * D, D), lambda k, w, l: (0, 0)),
                pl.BlockSpec((4 * D, D), lambda k, w, l: (0, 0)),
                pl.BlockSpec((1, 4 * D), lambda k, w, l: (0, 0)),
                pl.BlockSpec((2 * B, D), lambda k, w, l: (0, 0)),
            ],
            out_specs=[
                pl.BlockSpec((TC, B, 2 * D),
                             lambda k, w, l: (jnp.maximum(k - 1, 0), 0, 0)),
                pl.BlockSpec((1, D), lambda k, w, l: (0, 0)),
            ],
            scratch_shapes=[
                pltpu.VMEM((V, 2, 128), jnp.float32),
                pltpu.SemaphoreType.DMA((2,)),
                pltpu.VMEM((2 * S, 128), jnp.float32),
                pltpu.VMEM((2, M, 4 * D), jnp.float32),
                pltpu.VMEM((D, 4 * D), jnp.float32),
                pltpu.VMEM((D, 4 * D), jnp.float32),
                pltpu.VMEM((2 * B, D), jnp.float32),
                pltpu.VMEM((2 * B, D), jnp.float32),
            ],
        ),
        out_shape=[
            jax.ShapeDtypeStruct((T, B, 2 * D), jnp.float32),
            jax.ShapeDtypeStruct((1, D), jnp.float32),
        ],
        compiler_params=pltpu.CompilerParams(
            dimension_semantics=("arbitrary",),
            vmem_limit_bytes=56 * 1024 * 1024,
        ),
        name="bidir_lstm_encoder",
    )(words_flat, wrev_flat, emb, W_ih, W_hh, biasv, lenb2)

    return (finals, hall)


# speedup vs baseline: 14.2392x; 1.0004x over previous
import jax
import jax.numpy as jnp
from jax.experimental import pallas as pl
from jax.experimental.pallas import tpu as pltpu

B = 64
T = 512
D = 256
TC = 8
M = 2 * B * TC
S = M + 1


def _sig(x):
    return 0.5 * jnp.tanh(0.5 * x) + 0.5


def _gather_dot(words_ref, wrev_ref, emb_v, xs, gx_ref, wx, bias, chunk, sel):
    for s in range(TC):
        tt = chunk * TC + s
        for r in range(2 * B):
            if r < B:
                idx = words_ref[r * T + tt]
            else:
                idx = wrev_ref[(r - B) * T + tt]
            mi = s * 2 * B + r
            xs[mi:mi + 2 * S:S, :] = emb_v[idx]
    for j in range(M // 512):
        xj = jnp.concatenate([xs[pl.ds(j * 512, 512), :],
                              xs[pl.ds(S + j * 512, 512), :]], axis=-1)
        gx_ref[sel, pl.ds(j * 512, 512), :] = (
            jnp.dot(xj, wx, preferred_element_type=jnp.float32) + bias)


def _lstm_kernel(words_ref,
                 wrev_ref,
                 emb_hbm,
                 wih_ref,
                 whh_ref,
                 bias_ref,
                 lenb_ref,
                 hall_ref,
                 fin_ref,
                 emb_v,
                 copy_sem,
                 xs,
                 gx_ref,
                 wx_s, wh_s,
                 h_s, c_s):
    k = pl.program_id(0)
    nt = T // TC
    bias = bias_ref[...]
    lenb = lenb_ref[...]

    @pl.when(k == 0)
    def _init():
        c0 = pltpu.make_async_copy(emb_hbm.at[:, 0:128], emb_v.at[:, 0],
                                   copy_sem.at[0])
        c1 = pltpu.make_async_copy(emb_hbm.at[:, 128:256], emb_v.at[:, 1],
                                   copy_sem.at[1])
        c0.start()
        c1.start()
        wx_s[...] = wih_ref[...].T
        wh_s[...] = whh_ref[...].T
        h_s[...] = jnp.zeros_like(h_s)
        c_s[...] = jnp.zeros_like(c_s)
        c0.wait()
        c1.wait()

    sel = (k - 1) & 1
    tb = jnp.where(k > 0, (k - 1) * TC, 2 * T)
    wh = wh_s[...]
    for s in range(TC):
        t_idx = tb + s
        g = (jnp.dot(h_s[...], wh, preferred_element_type=jnp.float32)
             + gx_ref[sel, pl.ds(s * 2 * B, 2 * B), :])
        gi = g[:, 0:D]
        gf = g[:, D:2 * D]
        gg = g[:, 2 * D:3 * D]
        go = g[:, 3 * D:4 * D]
        c2 = _sig(gf) * c_s[...] + _sig(gi) * jnp.tanh(gg)
        h2 = _sig(go) * jnp.tanh(c2)
        m = lenb > t_idx
        hn = jnp.where(m, h2, h_s[...])
        h_s[...] = hn
        c_s[...] = jnp.where(m, c2, c_s[...])
        mo = jnp.where(m, hn, 0.0)
        hall_ref[s, :, 0:D] = mo[0:B]
        hall_ref[s, :, D:2 * D] = mo[B:2 * B]
    fin_ref[...] = h_s[2 * B - 1:2 * B, :]

    chunk = jnp.minimum(k, nt - 1)
    _gather_dot(words_ref, wrev_ref, emb_v, xs, gx_ref, wx_s[...], bias,
                chunk, k & 1)


def kernel(words, lengths, emb, W_ih, W_hh, b_ih, b_hh):
    lengths = lengths.astype(jnp.int32)
    words = words.astype(jnp.int32)
    words_flat = words.reshape(-1)
    idx_rev = jnp.clip(lengths[:, None] - 1 - jnp.arange(T)[None, :], 0)
    wrev_flat = jnp.take_along_axis(words, idx_rev, axis=1).reshape(-1)

    biasv = (b_ih + b_hh).reshape(1, 4 * D)
    lenb = jnp.broadcast_to(lengths[:, None], (B, D))
    lenb2 = jnp.concatenate([lenb, lenb], axis=0)

    V = emb.shape[0]
    nt = T // TC
    hall, finals = pl.pallas_call(
        _lstm_kernel,
        grid_spec=pltpu.PrefetchScalarGridSpec(
            num_scalar_prefetch=2,
            grid=(nt + 1,),
            in_specs=[
                pl.BlockSpec(memory_space=pl.ANY),
                pl.BlockSpec((4 * D, D), lambda k, w, l: (0, 0)),
                pl.BlockSpec((4 * D, D), lambda k, w, l: (0, 0)),
                pl.BlockSpec((1, 4 * D), lambda k, w, l: (0, 0)),
                pl.BlockSpec((2 * B, D), lambda k, w, l: (0, 0)),
            ],
            out_specs=[
                pl.BlockSpec((TC, B, 2 * D),
                             lambda k, w, l: (jnp.maximum(k - 1, 0), 0, 0)),
                pl.BlockSpec((1, D), lambda k, w, l: (0, 0)),
            ],
            scratch_shapes=[
                pltpu.VMEM((V, 2, 128), jnp.float32),
                pltpu.SemaphoreType.DMA((2,)),
                pltpu.VMEM((2 * S, 128), jnp.float32),
                pltpu.VMEM((2, M, 4 * D), jnp.float32),
                pltpu.VMEM((D, 4 * D), jnp.float32),
                pltpu.VMEM((D, 4 * D), jnp.float32),
                pltpu.VMEM((2 * B, D), jnp.float32),
                pltpu.VMEM((2 * B, D), jnp.float32),
            ],
        ),
        out_shape=[
            jax.ShapeDtypeStruct((T, B, 2 * D), jnp.float32),
            jax.ShapeDtypeStruct((1, D), jnp.float32),
        ],
        compiler_params=pltpu.CompilerParams(
            dimension_semantics=("arbitrary",),
            vmem_limit_bytes=56 * 1024 * 1024,
        ),
        name="bidir_lstm_encoder",
    )(words_flat, wrev_flat, emb, W_ih, W_hh, biasv, lenb2)

    return (finals, hall)
